# baseline (speedup 1.0000x reference)
import jax
import jax.numpy as jnp
from jax import lax
from jax.experimental import pallas as pl
from jax.experimental.pallas import tpu as pltpu
from jax.experimental.pallas import tpu_sc as plsc

N = 10000
E = 320000
DN = 128
DE = 16

NC = 2
NS = 16
NW = NC * NS
CH = 128
NCHUNK = E // CH
N_PAD = 10240
ROWS_PER_S = N_PAD // NS
DH = DN // 2

MC1 = NCHUNK // NS - (NCHUNK // NS) % 3
MC3 = NCHUNK // NW - (NCHUNK // NW) % 3

_ZERO16 = lambda: jnp.zeros((16,), jnp.float32)


def _zero_vmem_2d(ref, nrows, ncols):
    ngroups = ncols // 16

    def body(i, _):
        r = i // ngroups
        c = (i % ngroups) * 16
        ref[r, pl.ds(c, 16)] = _ZERO16()
        return 0

    lax.fori_loop(0, nrows * ngroups, body, 0)


def _agg_a_body(nf_lo, nf_hi, src, dst, sn_out, cnt_out,
                is0, is1, is2, id0, id1, id2, g0, g1, g2,
                msg_c, zb_n, zb_16, acc_n, acc_c,
                ld0, ld1, ld2, sg0, sg1, sg2, sc0, sc1, sc2):
    cid = lax.axis_index("c")
    sid = lax.axis_index("s")

    slots = ((is0, id0, g0, ld0, sg0, sc0),
             (is1, id1, g1, ld1, sg1, sc1),
             (is2, id2, g2, ld2, sg2, sc2))

    _zero_vmem_2d(zb_n, 128, DH)
    _zero_vmem_2d(zb_16, ROWS_PER_S, 16)
    one_hot = jnp.where(lax.iota(jnp.int32, 16) == 0,
                        jnp.float32(1.0), jnp.float32(0.0))

    def init_c(j, _):
        msg_c[j, :] = one_hot
        return 0
    lax.fori_loop(0, CH, init_c, 0)

    row0 = sid * ROWS_PER_S
    for q in range(5):
        pltpu.sync_copy(zb_n, acc_n.at[pl.ds(row0 + q * 128, 128)])
    pltpu.sync_copy(zb_16, acc_c.at[pl.ds(row0, ROWS_PER_S)])
    plsc.subcore_barrier()

    cb = sid * MC1

    def run_half(nf_half, with_cnt):
        def fire_loads(sl, i):
            eb = pl.multiple_of((cb + i) * CH, 8)
            pltpu.async_copy(src.at[pl.ds(eb, CH)], sl[0], sl[3])
            pltpu.async_copy(dst.at[pl.ds(eb, CH)], sl[1], sl[3])

        def wait_loads(sl):
            pltpu.make_async_copy(src.at[pl.ds(0, CH)], sl[0], sl[3]).wait()
            pltpu.make_async_copy(dst.at[pl.ds(0, CH)], sl[1], sl[3]).wait()

        def fire_gather(sl):
            pltpu.async_copy(nf_half.at[sl[0]], sl[2], sl[4])

        def wait_gather(sl):
            pltpu.make_async_copy(nf_half.at[sl[0]], sl[2], sl[4]).wait()

        def fire_scatters(sl):
            pltpu.async_copy(sl[2], acc_n.at[sl[1]], sl[5], add=True)
            if with_cnt:
                pltpu.async_copy(msg_c, acc_c.at[sl[1]], sl[5], add=True)

        def wait_scatters(sl):
            pltpu.make_async_copy(sl[2], acc_n.at[sl[1]], sl[5]).wait()
            if with_cnt:
                pltpu.make_async_copy(msg_c, acc_c.at[sl[1]], sl[5]).wait()

        fire_loads(slots[0], 0)
        fire_loads(slots[1], 1)
        wait_loads(slots[0])
        fire_gather(slots[0])

        NG = MC1 // 3

        def outer(g, _):
            for b in range(3):
                i = g * 3 + b
                sl = slots[b]
                sl_n = slots[(b + 1) % 3]
                sl_nn = slots[(b + 2) % 3]

                wait_gather(sl)
                fire_scatters(sl)

                if b < 2:
                    wait_loads(sl_n)
                    fire_gather(sl_n)
                else:
                    @pl.when(g < NG - 1)
                    def _():
                        wait_loads(sl_n)
                        fire_gather(sl_n)

                if b == 0:
                    @pl.when(g > 0)
                    def _():
                        wait_scatters(sl_nn)
                    fire_loads(sl_nn, i + 2)
                else:
                    @pl.when(g < NG - 1)
                    def _():
                        wait_scatters(sl_nn)
                        fire_loads(sl_nn, i + 2)
            return 0

        lax.fori_loop(0, NG, outer, 0)
        for b in range(3):
            wait_scatters(slots[b])

        @pl.when(sid < NCHUNK - NS * MC1)
        def _():
            eb = pl.multiple_of((NS * MC1 + sid) * CH, 8)
            sl = slots[0]
            pltpu.sync_copy(src.at[pl.ds(eb, CH)], sl[0])
            pltpu.sync_copy(dst.at[pl.ds(eb, CH)], sl[1])
            pltpu.async_copy(nf_half.at[sl[0]], sl[2], sl[4]).wait()
            pltpu.sync_copy(sl[2], acc_n.at[sl[1]], add=True)
            if with_cnt:
                pltpu.sync_copy(msg_c, acc_c.at[sl[1]], add=True)

    @pl.when(cid == 0)
    def _():
        run_half(nf_lo, False)

    @pl.when(cid == 1)
    def _():
        run_half(nf_hi, True)

    plsc.subcore_barrier()

    pltpu.sync_copy(acc_n.at[pl.ds(row0, ROWS_PER_S)],
                    sn_out.at[cid, pl.ds(row0, ROWS_PER_S)])

    @pl.when(cid == 1)
    def _():
        pltpu.sync_copy(acc_c.at[pl.ds(row0, ROWS_PER_S)],
                        cnt_out.at[pl.ds(row0, ROWS_PER_S)])


_agg_a = pl.kernel(
    _agg_a_body,
    out_type=(
        jax.ShapeDtypeStruct((NC, N_PAD, DH), jnp.float32),
        jax.ShapeDtypeStruct((N_PAD, 16), jnp.float32),
    ),
    mesh=plsc.VectorSubcoreMesh(core_axis_name="c", subcore_axis_name="s"),
    compiler_params=pltpu.CompilerParams(
        use_tc_tiling_on_sc=False, needs_layout_passes=False),
    scratch_types=(
        [pltpu.VMEM((CH,), jnp.int32)] * 6
        + [pltpu.VMEM((CH, DH), jnp.float32)] * 3
        + [pltpu.VMEM((CH, 16), jnp.float32),
           pltpu.VMEM((128, DH), jnp.float32),
           pltpu.VMEM((ROWS_PER_S, 16), jnp.float32),
           pltpu.VMEM_SHARED((N_PAD, DH), jnp.float32),
           pltpu.VMEM_SHARED((N_PAD, 16), jnp.float32)]
        + [pltpu.SemaphoreType.DMA] * 9
    ),
)


def _agg_b_body(efeats, dst, se_out,
                id0, id1, id2, e0, e1, e2, zb_16, acc_e,
                ld0, ld1, ld2, sc0, sc1, sc2):
    cid = lax.axis_index("c")
    sid = lax.axis_index("s")
    w = cid * NS + sid

    _zero_vmem_2d(zb_16, ROWS_PER_S, 16)
    row0 = sid * ROWS_PER_S
    pltpu.sync_copy(zb_16, acc_e.at[pl.ds(row0, ROWS_PER_S)])
    plsc.subcore_barrier()

    cb = w * MC3
    slots = ((id0, e0, ld0, sc0), (id1, e1, ld1, sc1), (id2, e2, ld2, sc2))

    def fire_loads(sl, i):
        eb = pl.multiple_of((cb + i) * CH, 8)
        pltpu.async_copy(dst.at[pl.ds(eb, CH)], sl[0], sl[2])
        pltpu.async_copy(efeats.at[pl.ds(eb, CH)], sl[1], sl[2])

    def wait_loads(sl):
        pltpu.make_async_copy(dst.at[pl.ds(0, CH)], sl[0], sl[2]).wait()
        pltpu.make_async_copy(efeats.at[pl.ds(0, CH)], sl[1], sl[2]).wait()

    def fire_scatter(sl):
        pltpu.async_copy(sl[1], acc_e.at[sl[0]], sl[3], add=True)

    def wait_scatter(sl):
        pltpu.make_async_copy(sl[1], acc_e.at[sl[0]], sl[3]).wait()

    fire_loads(slots[0], 0)
    fire_loads(slots[1], 1)

    NG = MC3 // 3

    def outer(g, _):
        for b in range(3):
            i = g * 3 + b
            sl = slots[b]
            sl_nn = slots[(b + 2) % 3]

            wait_loads(sl)
            fire_scatter(sl)

            if b == 0:
                @pl.when(g > 0)
                def _():
                    wait_scatter(sl_nn)
                fire_loads(sl_nn, i + 2)
            else:
                @pl.when(g < NG - 1)
                def _():
                    wait_scatter(sl_nn)
                    fire_loads(sl_nn, i + 2)
        return 0

    lax.fori_loop(0, NG, outer, 0)
    for b in range(3):
        wait_scatter(slots[b])

    @pl.when(w < NCHUNK - NW * MC3)
    def _():
        eb = pl.multiple_of((NW * MC3 + w) * CH, 8)
        sl = slots[0]
        pltpu.sync_copy(dst.at[pl.ds(eb, CH)], sl[0])
        pltpu.sync_copy(efeats.at[pl.ds(eb, CH)], sl[1])
        pltpu.sync_copy(sl[1], acc_e.at[sl[0]], add=True)

    plsc.subcore_barrier()
    pltpu.sync_copy(acc_e.at[pl.ds(row0, ROWS_PER_S)],
                    se_out.at[cid, pl.ds(row0, ROWS_PER_S)])


_agg_b = pl.kernel(
    _agg_b_body,
    out_type=jax.ShapeDtypeStruct((NC, N_PAD, 16), jnp.float32),
    mesh=plsc.VectorSubcoreMesh(core_axis_name="c", subcore_axis_name="s"),
    compiler_params=pltpu.CompilerParams(
        use_tc_tiling_on_sc=False, needs_layout_passes=False),
    scratch_types=(
        [pltpu.VMEM((CH,), jnp.int32)] * 3
        + [pltpu.VMEM((CH, DE), jnp.float32)] * 3
        + [pltpu.VMEM((ROWS_PER_S, 16), jnp.float32),
           pltpu.VMEM_SHARED((N_PAD, 16), jnp.float32)]
        + [pltpu.SemaphoreType.DMA] * 6
    ),
)


def _node_mlp_body(snp, sep, cntp, nf, w1t, w2t, w3t, b, wst, wdt, be,
                   h_ref, ps_ref, pd_ref):
    sn = jnp.concatenate([snp[0], snp[1]], axis=1)
    se = sep[0] + sep[1]
    cnt = cntp[:, 0:1]
    inv = 1.0 / jnp.maximum(cnt, 1.0)
    t = (jnp.dot(sn, w2t[...], preferred_element_type=jnp.float32)
         + jnp.dot(se, w3t[...], preferred_element_type=jnp.float32))
    h = jnp.maximum(
        jnp.dot(nf[...], w1t[...], preferred_element_type=jnp.float32)
        + t * inv + b[...], 0.0)
    h_ref[...] = h
    ps_ref[...] = jnp.dot(h, wst[...], preferred_element_type=jnp.float32) + be[...]
    pd_ref[...] = jnp.dot(h, wdt[...], preferred_element_type=jnp.float32)


_BLK = 400
_node_mlp = pl.pallas_call(
    _node_mlp_body,
    grid=(N // _BLK,),
    in_specs=[
        pl.BlockSpec((NC, _BLK, DH), lambda i: (0, i, 0)),
        pl.BlockSpec((NC, _BLK, 16), lambda i: (0, i, 0)),
        pl.BlockSpec((_BLK, 16), lambda i: (i, 0)),
        pl.BlockSpec((_BLK, DN), lambda i: (i, 0)),
        pl.BlockSpec((DN, DN), lambda i: (0, 0)),
        pl.BlockSpec((DN, DN), lambda i: (0, 0)),
        pl.BlockSpec((DE, DN), lambda i: (0, 0)),
        pl.BlockSpec((1, DN), lambda i: (0, 0)),
        pl.BlockSpec((DN, DE), lambda i: (0, 0)),
        pl.BlockSpec((DN, DE), lambda i: (0, 0)),
        pl.BlockSpec((1, DE), lambda i: (0, 0)),
    ],
    out_specs=[
        pl.BlockSpec((_BLK, DN), lambda i: (i, 0)),
        pl.BlockSpec((_BLK, DE), lambda i: (i, 0)),
        pl.BlockSpec((_BLK, DE), lambda i: (i, 0)),
    ],
    out_shape=[
        jax.ShapeDtypeStruct((N, DN), jnp.float32),
        jax.ShapeDtypeStruct((N, DE), jnp.float32),
        jax.ShapeDtypeStruct((N, DE), jnp.float32),
    ],
)


def _edge_body(psrc, pdst, src, dst, out,
               is0, is1, is2, id0, id1, id2, a0, a1, a2, b0, b1, b2,
               o0, o1, o2, ld0, ld1, ld2, sg0, sg1, sg2, st0, st1, st2):
    cid = lax.axis_index("c")
    sid = lax.axis_index("s")
    w = cid * NS + sid
    cb = w * MC3

    slots = ((is0, id0, a0, b0, o0, ld0, sg0, st0),
             (is1, id1, a1, b1, o1, ld1, sg1, st1),
             (is2, id2, a2, b2, o2, ld2, sg2, st2))

    def fire_loads(sl, i):
        eb = pl.multiple_of((cb + i) * CH, 8)
        pltpu.async_copy(src.at[pl.ds(eb, CH)], sl[0], sl[5])
        pltpu.async_copy(dst.at[pl.ds(eb, CH)], sl[1], sl[5])

    def wait_loads(sl):
        pltpu.make_async_copy(src.at[pl.ds(0, CH)], sl[0], sl[5]).wait()
        pltpu.make_async_copy(dst.at[pl.ds(0, CH)], sl[1], sl[5]).wait()

    def fire_gathers(sl):
        pltpu.async_copy(psrc.at[sl[0]], sl[2], sl[6])
        pltpu.async_copy(pdst.at[sl[1]], sl[3], sl[6])

    def wait_gathers(sl):
        pltpu.make_async_copy(psrc.at[sl[0]], sl[2], sl[6]).wait()
        pltpu.make_async_copy(pdst.at[sl[1]], sl[3], sl[6]).wait()

    def compute(sl):
        def body(j, _):
            for r in range(4):
                jj = j * 4 + r
                v = jnp.maximum(sl[2][jj, :] + sl[3][jj, :], 0.0)
                q, off = jj // 8, (jj % 8) * 16
                sl[4][q, pl.ds(off, 16)] = v
            return 0
        lax.fori_loop(0, CH // 4, body, 0)

    def fire_store(sl, i):
        rb = pl.multiple_of((cb + i) * (CH // 8), 8)
        pltpu.async_copy(sl[4], out.at[pl.ds(rb, CH // 8)], sl[7])

    def wait_store(sl):
        pltpu.make_async_copy(sl[4], out.at[pl.ds(0, CH // 8)], sl[7]).wait()

    fire_loads(slots[0], 0)
    fire_loads(slots[1], 1)
    wait_loads(slots[0])
    fire_gathers(slots[0])

    NG = MC3 // 3

    def outer(g, _):
        for b in range(3):
            i = g * 3 + b
            sl = slots[b]
            sl_n = slots[(b + 1) % 3]
            sl_nn = slots[(b + 2) % 3]

            wait_gathers(sl)

            @pl.when(g > 0)
            def _():
                wait_store(sl)
            compute(sl)
            fire_store(sl, i)

            if b < 2:
                wait_loads(sl_n)
                fire_gathers(sl_n)
            else:
                @pl.when(g < NG - 1)
                def _():
                    wait_loads(sl_n)
                    fire_gathers(sl_n)

            if b == 0:
                fire_loads(sl_nn, i + 2)
            else:
                @pl.when(g < NG - 1)
                def _():
                    fire_loads(sl_nn, i + 2)
        return 0

    lax.fori_loop(0, NG, outer, 0)
    for b in range(3):
        wait_store(slots[b])

    @pl.when(w < NCHUNK - NW * MC3)
    def _():
        eb = pl.multiple_of((NW * MC3 + w) * CH, 8)
        sl = slots[0]
        pltpu.sync_copy(src.at[pl.ds(eb, CH)], sl[0])
        pltpu.sync_copy(dst.at[pl.ds(eb, CH)], sl[1])
        ca = pltpu.async_copy(psrc.at[sl[0]], sl[2], sl[6])
        cbd = pltpu.async_copy(pdst.at[sl[1]], sl[3], sl[6])
        ca.wait()
        cbd.wait()
        compute(sl)
        rb = pl.multiple_of((NW * MC3 + w) * (CH // 8), 8)
        pltpu.sync_copy(sl[4], out.at[pl.ds(rb, CH // 8)])


_edge_mlp = pl.kernel(
    _edge_body,
    out_type=jax.ShapeDtypeStruct((E // 8, 128), jnp.float32),
    mesh=plsc.VectorSubcoreMesh(core_axis_name="c", subcore_axis_name="s"),
    compiler_params=pltpu.CompilerParams(
        use_tc_tiling_on_sc=False, needs_layout_passes=False),
    scratch_types=(
        [pltpu.VMEM((CH,), jnp.int32)] * 6
        + [pltpu.VMEM((CH, DE), jnp.float32)] * 6
        + [pltpu.VMEM((CH // 8, 128), jnp.float32)] * 3
        + [pltpu.SemaphoreType.DMA] * 9
    ),
)


def kernel(nfeats, efeats, edge_index, W_apply_w, W_apply_b, W_edge_w, W_edge_b):
    src = edge_index[0].astype(jnp.int32)
    dst = edge_index[1].astype(jnp.int32)

    sn_p, cnt_p = _agg_a(nfeats[:, :DH], nfeats[:, DH:], src, dst)
    se_p = _agg_b(efeats, dst)

    w1t = W_apply_w[:, :DN].T
    w2t = W_apply_w[:, DN:2 * DN].T
    w3t = W_apply_w[:, 2 * DN:].T
    wst = W_edge_w[:, :DN].T
    wdt = W_edge_w[:, DN:].T
    h_nodes, psrc, pdst = _node_mlp(
        sn_p, se_p, cnt_p, nfeats, w1t, w2t, w3t, W_apply_b[None, :],
        wst, wdt, W_edge_b[None, :])

    h_edges4 = _edge_mlp(psrc, pdst, src, dst)
    return (h_nodes, h_edges4.reshape(E, DE))


# speedup vs baseline: 1.2310x; 1.2310x over previous
import jax
import jax.numpy as jnp
from jax import lax
from jax.experimental import pallas as pl
from jax.experimental.pallas import tpu as pltpu
from jax.experimental.pallas import tpu_sc as plsc

N = 10000
E = 320000
DN = 128
DE = 16

NC = 2
NS = 16
NW = NC * NS
CH = 128
NCHUNK = E // CH
N_PAD = 10240
ROWS_PER_S = N_PAD // NS
DH = DN // 2

MC1 = NCHUNK // NS - (NCHUNK // NS) % 3
MC3 = NCHUNK // NW - (NCHUNK // NW) % 3

_ZERO16 = lambda: jnp.zeros((16,), jnp.float32)


def _zero_vmem_2d(ref, nrows, ncols):
    ngroups = ncols // 16

    def body(i, _):
        r = i // ngroups
        c = (i % ngroups) * 16
        ref[r, pl.ds(c, 16)] = _ZERO16()
        return 0

    lax.fori_loop(0, nrows * ngroups, body, 0)


def _agg_a_body(nf_lo, nf_hi, src, dst, sn_out, cnt_out,
                is0, is1, is2, is3, is4, is5,
                id0, id1, id2, id3, id4, id5,
                g0, g1, g2, g3, g4, g5,
                msg_c, zb_n, zb_16, acc_n, acc_c,
                ld0, ld1, ld2, ld3, ld4, ld5,
                sg0, sg1, sg2, sg3, sg4, sg5,
                sc0, sc1, sc2, sc3, sc4, sc5):
    cid = lax.axis_index("c")
    sid = lax.axis_index("s")

    slots = ((is0, id0, g0, ld0, sg0, sc0),
             (is1, id1, g1, ld1, sg1, sc1),
             (is2, id2, g2, ld2, sg2, sc2),
             (is3, id3, g3, ld3, sg3, sc3),
             (is4, id4, g4, ld4, sg4, sc4),
             (is5, id5, g5, ld5, sg5, sc5))
    NSL = 6

    _zero_vmem_2d(zb_n, 128, DH)
    _zero_vmem_2d(zb_16, ROWS_PER_S, 16)
    one_hot = jnp.where(lax.iota(jnp.int32, 16) == 0,
                        jnp.float32(1.0), jnp.float32(0.0))

    def init_c(j, _):
        msg_c[j, :] = one_hot
        return 0
    lax.fori_loop(0, CH, init_c, 0)

    row0 = sid * ROWS_PER_S
    for q in range(5):
        pltpu.sync_copy(zb_n, acc_n.at[pl.ds(row0 + q * 128, 128)])
    pltpu.sync_copy(zb_16, acc_c.at[pl.ds(row0, ROWS_PER_S)])
    plsc.subcore_barrier()

    cb = sid * MC1

    def run_half(nf_half, with_cnt):
        def fire_loads(sl, i):
            eb = pl.multiple_of((cb + i) * CH, 8)
            pltpu.async_copy(src.at[pl.ds(eb, CH)], sl[0], sl[3])
            pltpu.async_copy(dst.at[pl.ds(eb, CH)], sl[1], sl[3])

        def wait_loads(sl):
            pltpu.make_async_copy(src.at[pl.ds(0, CH)], sl[0], sl[3]).wait()
            pltpu.make_async_copy(dst.at[pl.ds(0, CH)], sl[1], sl[3]).wait()

        def fire_gather(sl):
            pltpu.async_copy(nf_half.at[sl[0]], sl[2], sl[4])

        def wait_gather(sl):
            pltpu.make_async_copy(nf_half.at[sl[0]], sl[2], sl[4]).wait()

        def fire_scatters(sl):
            pltpu.async_copy(sl[2], acc_n.at[sl[1]], sl[5], add=True)
            if with_cnt:
                pltpu.async_copy(msg_c, acc_c.at[sl[1]], sl[5], add=True)

        def wait_scatters(sl):
            pltpu.make_async_copy(sl[2], acc_n.at[sl[1]], sl[5]).wait()
            if with_cnt:
                pltpu.make_async_copy(msg_c, acc_c.at[sl[1]], sl[5]).wait()

        for i in range(4):
            fire_loads(slots[i], i)
        for i in range(2):
            wait_loads(slots[i])
            fire_gather(slots[i])

        NG = MC1 // NSL

        def outer(g, _):
            for b in range(NSL):
                i = g * NSL + b
                sl = slots[b]
                sl_b2 = slots[(b + 2) % NSL]
                sl_b4 = slots[(b + 4) % NSL]

                wait_gather(sl)
                fire_scatters(sl)

                if b < 4:
                    wait_loads(sl_b2)
                    fire_gather(sl_b2)
                else:
                    @pl.when(g < NG - 1)
                    def _():
                        wait_loads(sl_b2)
                        fire_gather(sl_b2)

                if b < 2:
                    @pl.when(g > 0)
                    def _():
                        wait_scatters(sl_b4)
                    fire_loads(sl_b4, i + 4)
                else:
                    @pl.when(g < NG - 1)
                    def _():
                        wait_scatters(sl_b4)
                        fire_loads(sl_b4, i + 4)
            return 0

        lax.fori_loop(0, NG, outer, 0)
        for b in range(NSL):
            wait_scatters(slots[b])

        @pl.when(sid < NCHUNK - NS * MC1)
        def _():
            eb = pl.multiple_of((NS * MC1 + sid) * CH, 8)
            sl = slots[0]
            pltpu.sync_copy(src.at[pl.ds(eb, CH)], sl[0])
            pltpu.sync_copy(dst.at[pl.ds(eb, CH)], sl[1])
            pltpu.async_copy(nf_half.at[sl[0]], sl[2], sl[4]).wait()
            pltpu.sync_copy(sl[2], acc_n.at[sl[1]], add=True)
            if with_cnt:
                pltpu.sync_copy(msg_c, acc_c.at[sl[1]], add=True)

    @pl.when(cid == 0)
    def _():
        run_half(nf_lo, False)

    @pl.when(cid == 1)
    def _():
        run_half(nf_hi, True)

    plsc.subcore_barrier()

    pltpu.sync_copy(acc_n.at[pl.ds(row0, ROWS_PER_S)],
                    sn_out.at[cid, pl.ds(row0, ROWS_PER_S)])

    @pl.when(cid == 1)
    def _():
        pltpu.sync_copy(acc_c.at[pl.ds(row0, ROWS_PER_S)],
                        cnt_out.at[pl.ds(row0, ROWS_PER_S)])


_agg_a = pl.kernel(
    _agg_a_body,
    out_type=(
        jax.ShapeDtypeStruct((NC, N_PAD, DH), jnp.float32),
        jax.ShapeDtypeStruct((N_PAD, 16), jnp.float32),
    ),
    mesh=plsc.VectorSubcoreMesh(core_axis_name="c", subcore_axis_name="s"),
    compiler_params=pltpu.CompilerParams(
        use_tc_tiling_on_sc=False, needs_layout_passes=False),
    scratch_types=(
        [pltpu.VMEM((CH,), jnp.int32)] * 12
        + [pltpu.VMEM((CH, DH), jnp.float32)] * 6
        + [pltpu.VMEM((CH, 16), jnp.float32),
           pltpu.VMEM((128, DH), jnp.float32),
           pltpu.VMEM((ROWS_PER_S, 16), jnp.float32),
           pltpu.VMEM_SHARED((N_PAD, DH), jnp.float32),
           pltpu.VMEM_SHARED((N_PAD, 16), jnp.float32)]
        + [pltpu.SemaphoreType.DMA] * 18
    ),
)


def _agg_b_body(efeats, dst, se_out,
                id0, id1, id2, e0, e1, e2, zb_16, acc_e,
                ld0, ld1, ld2, sc0, sc1, sc2):
    cid = lax.axis_index("c")
    sid = lax.axis_index("s")
    w = cid * NS + sid

    _zero_vmem_2d(zb_16, ROWS_PER_S, 16)
    row0 = sid * ROWS_PER_S
    pltpu.sync_copy(zb_16, acc_e.at[pl.ds(row0, ROWS_PER_S)])
    plsc.subcore_barrier()

    cb = w * MC3
    slots = ((id0, e0, ld0, sc0), (id1, e1, ld1, sc1), (id2, e2, ld2, sc2))

    def fire_loads(sl, i):
        eb = pl.multiple_of((cb + i) * CH, 8)
        pltpu.async_copy(dst.at[pl.ds(eb, CH)], sl[0], sl[2])
        pltpu.async_copy(efeats.at[pl.ds(eb, CH)], sl[1], sl[2])

    def wait_loads(sl):
        pltpu.make_async_copy(dst.at[pl.ds(0, CH)], sl[0], sl[2]).wait()
        pltpu.make_async_copy(efeats.at[pl.ds(0, CH)], sl[1], sl[2]).wait()

    def fire_scatter(sl):
        pltpu.async_copy(sl[1], acc_e.at[sl[0]], sl[3], add=True)

    def wait_scatter(sl):
        pltpu.make_async_copy(sl[1], acc_e.at[sl[0]], sl[3]).wait()

    fire_loads(slots[0], 0)
    fire_loads(slots[1], 1)

    NG = MC3 // 3

    def outer(g, _):
        for b in range(3):
            i = g * 3 + b
            sl = slots[b]
            sl_nn = slots[(b + 2) % 3]

            wait_loads(sl)
            fire_scatter(sl)

            if b == 0:
                @pl.when(g > 0)
                def _():
                    wait_scatter(sl_nn)
                fire_loads(sl_nn, i + 2)
            else:
                @pl.when(g < NG - 1)
                def _():
                    wait_scatter(sl_nn)
                    fire_loads(sl_nn, i + 2)
        return 0

    lax.fori_loop(0, NG, outer, 0)
    for b in range(3):
        wait_scatter(slots[b])

    @pl.when(w < NCHUNK - NW * MC3)
    def _():
        eb = pl.multiple_of((NW * MC3 + w) * CH, 8)
        sl = slots[0]
        pltpu.sync_copy(dst.at[pl.ds(eb, CH)], sl[0])
        pltpu.sync_copy(efeats.at[pl.ds(eb, CH)], sl[1])
        pltpu.sync_copy(sl[1], acc_e.at[sl[0]], add=True)

    plsc.subcore_barrier()
    pltpu.sync_copy(acc_e.at[pl.ds(row0, ROWS_PER_S)],
                    se_out.at[cid, pl.ds(row0, ROWS_PER_S)])


_agg_b = pl.kernel(
    _agg_b_body,
    out_type=jax.ShapeDtypeStruct((NC, N_PAD, 16), jnp.float32),
    mesh=plsc.VectorSubcoreMesh(core_axis_name="c", subcore_axis_name="s"),
    compiler_params=pltpu.CompilerParams(
        use_tc_tiling_on_sc=False, needs_layout_passes=False),
    scratch_types=(
        [pltpu.VMEM((CH,), jnp.int32)] * 3
        + [pltpu.VMEM((CH, DE), jnp.float32)] * 3
        + [pltpu.VMEM((ROWS_PER_S, 16), jnp.float32),
           pltpu.VMEM_SHARED((N_PAD, 16), jnp.float32)]
        + [pltpu.SemaphoreType.DMA] * 6
    ),
)


def _node_mlp_body(snp, sep, cntp, nf, w1t, w2t, w3t, b, wst, wdt, be,
                   h_ref, ps_ref, pd_ref):
    sn = jnp.concatenate([snp[0], snp[1]], axis=1)
    se = sep[0] + sep[1]
    cnt = cntp[:, 0:1]
    inv = 1.0 / jnp.maximum(cnt, 1.0)
    t = (jnp.dot(sn, w2t[...], preferred_element_type=jnp.float32)
         + jnp.dot(se, w3t[...], preferred_element_type=jnp.float32))
    h = jnp.maximum(
        jnp.dot(nf[...], w1t[...], preferred_element_type=jnp.float32)
        + t * inv + b[...], 0.0)
    h_ref[...] = h
    ps_ref[...] = jnp.dot(h, wst[...], preferred_element_type=jnp.float32) + be[...]
    pd_ref[...] = jnp.dot(h, wdt[...], preferred_element_type=jnp.float32)


_BLK = 400
_node_mlp = pl.pallas_call(
    _node_mlp_body,
    grid=(N // _BLK,),
    in_specs=[
        pl.BlockSpec((NC, _BLK, DH), lambda i: (0, i, 0)),
        pl.BlockSpec((NC, _BLK, 16), lambda i: (0, i, 0)),
        pl.BlockSpec((_BLK, 16), lambda i: (i, 0)),
        pl.BlockSpec((_BLK, DN), lambda i: (i, 0)),
        pl.BlockSpec((DN, DN), lambda i: (0, 0)),
        pl.BlockSpec((DN, DN), lambda i: (0, 0)),
        pl.BlockSpec((DE, DN), lambda i: (0, 0)),
        pl.BlockSpec((1, DN), lambda i: (0, 0)),
        pl.BlockSpec((DN, DE), lambda i: (0, 0)),
        pl.BlockSpec((DN, DE), lambda i: (0, 0)),
        pl.BlockSpec((1, DE), lambda i: (0, 0)),
    ],
    out_specs=[
        pl.BlockSpec((_BLK, DN), lambda i: (i, 0)),
        pl.BlockSpec((_BLK, DE), lambda i: (i, 0)),
        pl.BlockSpec((_BLK, DE), lambda i: (i, 0)),
    ],
    out_shape=[
        jax.ShapeDtypeStruct((N, DN), jnp.float32),
        jax.ShapeDtypeStruct((N, DE), jnp.float32),
        jax.ShapeDtypeStruct((N, DE), jnp.float32),
    ],
)


def _edge_body(psrc, pdst, src, dst, out,
               is0, is1, is2, id0, id1, id2, a0, a1, a2, b0, b1, b2,
               o0, o1, o2, ld0, ld1, ld2, sg0, sg1, sg2, st0, st1, st2):
    cid = lax.axis_index("c")
    sid = lax.axis_index("s")
    w = cid * NS + sid
    cb = w * MC3

    slots = ((is0, id0, a0, b0, o0, ld0, sg0, st0),
             (is1, id1, a1, b1, o1, ld1, sg1, st1),
             (is2, id2, a2, b2, o2, ld2, sg2, st2))

    def fire_loads(sl, i):
        eb = pl.multiple_of((cb + i) * CH, 8)
        pltpu.async_copy(src.at[pl.ds(eb, CH)], sl[0], sl[5])
        pltpu.async_copy(dst.at[pl.ds(eb, CH)], sl[1], sl[5])

    def wait_loads(sl):
        pltpu.make_async_copy(src.at[pl.ds(0, CH)], sl[0], sl[5]).wait()
        pltpu.make_async_copy(dst.at[pl.ds(0, CH)], sl[1], sl[5]).wait()

    def fire_gathers(sl):
        pltpu.async_copy(psrc.at[sl[0]], sl[2], sl[6])
        pltpu.async_copy(pdst.at[sl[1]], sl[3], sl[6])

    def wait_gathers(sl):
        pltpu.make_async_copy(psrc.at[sl[0]], sl[2], sl[6]).wait()
        pltpu.make_async_copy(pdst.at[sl[1]], sl[3], sl[6]).wait()

    def compute(sl):
        zeros16 = jnp.zeros((16,), jnp.int32)

        def body(j, iv):
            for r in range(4):
                jj = j * 4 + r
                v = jnp.maximum(sl[2][jj, :] + sl[3][jj, :], 0.0)
                plsc.store_scatter(sl[4], [zeros16, iv], v)
                iv = iv + 1
            return iv
        lax.fori_loop(0, CH // 4, body, lax.iota(jnp.int32, 16) * CH)

    def fire_store(sl, i):
        eb = pl.multiple_of((cb + i) * CH, 8)
        pltpu.async_copy(sl[4], out.at[:, pl.ds(eb, CH)], sl[7])

    def wait_store(sl):
        pltpu.make_async_copy(sl[4], out.at[:, pl.ds(0, CH)], sl[7]).wait()

    fire_loads(slots[0], 0)
    fire_loads(slots[1], 1)
    wait_loads(slots[0])
    fire_gathers(slots[0])

    NG = MC3 // 3

    def outer(g, _):
        for b in range(3):
            i = g * 3 + b
            sl = slots[b]
            sl_n = slots[(b + 1) % 3]
            sl_nn = slots[(b + 2) % 3]

            wait_gathers(sl)

            @pl.when(g > 0)
            def _():
                wait_store(sl)
            compute(sl)
            fire_store(sl, i)

            if b < 2:
                wait_loads(sl_n)
                fire_gathers(sl_n)
            else:
                @pl.when(g < NG - 1)
                def _():
                    wait_loads(sl_n)
                    fire_gathers(sl_n)

            if b == 0:
                fire_loads(sl_nn, i + 2)
            else:
                @pl.when(g < NG - 1)
                def _():
                    fire_loads(sl_nn, i + 2)
        return 0

    lax.fori_loop(0, NG, outer, 0)
    for b in range(3):
        wait_store(slots[b])

    @pl.when(w < NCHUNK - NW * MC3)
    def _():
        eb = pl.multiple_of((NW * MC3 + w) * CH, 8)
        sl = slots[0]
        pltpu.sync_copy(src.at[pl.ds(eb, CH)], sl[0])
        pltpu.sync_copy(dst.at[pl.ds(eb, CH)], sl[1])
        ca = pltpu.async_copy(psrc.at[sl[0]], sl[2], sl[6])
        cbd = pltpu.async_copy(pdst.at[sl[1]], sl[3], sl[6])
        ca.wait()
        cbd.wait()
        compute(sl)
        pltpu.sync_copy(sl[4], out.at[:, pl.ds(eb, CH)])


_edge_mlp = pl.kernel(
    _edge_body,
    out_type=jax.ShapeDtypeStruct((DE, E), jnp.float32),
    mesh=plsc.VectorSubcoreMesh(core_axis_name="c", subcore_axis_name="s"),
    compiler_params=pltpu.CompilerParams(
        use_tc_tiling_on_sc=False, needs_layout_passes=False),
    scratch_types=(
        [pltpu.VMEM((CH,), jnp.int32)] * 6
        + [pltpu.VMEM((CH, DE), jnp.float32)] * 6
        + [pltpu.VMEM((DE, CH), jnp.float32)] * 3
        + [pltpu.SemaphoreType.DMA] * 9
    ),
)


def kernel(nfeats, efeats, edge_index, W_apply_w, W_apply_b, W_edge_w, W_edge_b):
    src = edge_index[0].astype(jnp.int32)
    dst = edge_index[1].astype(jnp.int32)

    sn_p, cnt_p = _agg_a(nfeats[:, :DH], nfeats[:, DH:], src, dst)
    se_p = _agg_b(efeats, dst)

    w1t = W_apply_w[:, :DN].T
    w2t = W_apply_w[:, DN:2 * DN].T
    w3t = W_apply_w[:, 2 * DN:].T
    wst = W_edge_w[:, :DN].T
    wdt = W_edge_w[:, DN:].T
    h_nodes, psrc, pdst = _node_mlp(
        sn_p, se_p, cnt_p, nfeats, w1t, w2t, w3t, W_apply_b[None, :],
        wst, wdt, W_edge_b[None, :])

    h_edges_t = _edge_mlp(psrc, pdst, src, dst)
    return (h_nodes, h_edges_t.T)


# speedup vs baseline: 1.4461x; 1.1747x over previous
import jax
import jax.numpy as jnp
from jax import lax
from jax.experimental import pallas as pl
from jax.experimental.pallas import tpu as pltpu
from jax.experimental.pallas import tpu_sc as plsc

N = 10000
E = 320000
DN = 128
DE = 16

NC = 2
NS = 16
NW = NC * NS
CH = 128
NCHUNK = E // CH
N_PAD = 10240
ROWS_PER_S = N_PAD // NS
DH = DN // 2

MC1 = NCHUNK // NS - (NCHUNK // NS) % 3
MC3 = NCHUNK // NW - (NCHUNK // NW) % 3

_ZERO16 = lambda: jnp.zeros((16,), jnp.float32)


def _zero_vmem_2d(ref, nrows, ncols):
    ngroups = ncols // 16

    def body(i, _):
        r = i // ngroups
        c = (i % ngroups) * 16
        ref[r, pl.ds(c, 16)] = _ZERO16()
        return 0

    lax.fori_loop(0, nrows * ngroups, body, 0)


def _agg_a_body(nf_lo, nf_hi, src, dst, sn_out, cnt_out,
                is0, is1, is2, is3, is4, is5,
                id0, id1, id2, id3, id4, id5,
                g0, g1, g2, g3, g4, g5,
                msg_c, zb_n, zb_16, acc_n, acc_c,
                ld0, ld1, ld2, ld3, ld4, ld5,
                sg0, sg1, sg2, sg3, sg4, sg5,
                sc0, sc1, sc2, sc3, sc4, sc5):
    cid = lax.axis_index("c")
    sid = lax.axis_index("s")

    slots = ((is0, id0, g0, ld0, sg0, sc0),
             (is1, id1, g1, ld1, sg1, sc1),
             (is2, id2, g2, ld2, sg2, sc2),
             (is3, id3, g3, ld3, sg3, sc3),
             (is4, id4, g4, ld4, sg4, sc4),
             (is5, id5, g5, ld5, sg5, sc5))
    NSL = 6

    _zero_vmem_2d(zb_n, 128, DH)
    _zero_vmem_2d(zb_16, ROWS_PER_S, 16)
    one_hot = jnp.where(lax.iota(jnp.int32, 16) == 0,
                        jnp.float32(1.0), jnp.float32(0.0))

    def init_c(j, _):
        msg_c[j, :] = one_hot
        return 0
    lax.fori_loop(0, CH, init_c, 0)

    row0 = sid * ROWS_PER_S
    for q in range(5):
        pltpu.sync_copy(zb_n, acc_n.at[pl.ds(row0 + q * 128, 128)])
    pltpu.sync_copy(zb_16, acc_c.at[pl.ds(row0, ROWS_PER_S)])
    plsc.subcore_barrier()

    cb = sid * MC1

    def run_half(nf_half, with_cnt):
        def fire_loads(sl, i):
            eb = pl.multiple_of((cb + i) * CH, 8)
            pltpu.async_copy(src.at[pl.ds(eb, CH)], sl[0], sl[3])
            pltpu.async_copy(dst.at[pl.ds(eb, CH)], sl[1], sl[3])

        def wait_loads(sl):
            pltpu.make_async_copy(src.at[pl.ds(0, CH)], sl[0], sl[3]).wait()
            pltpu.make_async_copy(dst.at[pl.ds(0, CH)], sl[1], sl[3]).wait()

        def fire_gather(sl):
            pltpu.async_copy(nf_half.at[sl[0]], sl[2], sl[4])

        def wait_gather(sl):
            pltpu.make_async_copy(nf_half.at[sl[0]], sl[2], sl[4]).wait()

        def fire_scatters(sl):
            pltpu.async_copy(sl[2], acc_n.at[sl[1]], sl[5], add=True)
            if with_cnt:
                pltpu.async_copy(msg_c, acc_c.at[sl[1]], sl[5], add=True)

        def wait_scatters(sl):
            pltpu.make_async_copy(sl[2], acc_n.at[sl[1]], sl[5]).wait()
            if with_cnt:
                pltpu.make_async_copy(msg_c, acc_c.at[sl[1]], sl[5]).wait()

        for i in range(4):
            fire_loads(slots[i], i)
        for i in range(2):
            wait_loads(slots[i])
            fire_gather(slots[i])

        NG = MC1 // NSL

        def outer(g, _):
            for b in range(NSL):
                i = g * NSL + b
                sl = slots[b]
                sl_b2 = slots[(b + 2) % NSL]
                sl_b4 = slots[(b + 4) % NSL]

                wait_gather(sl)
                fire_scatters(sl)

                if b < 4:
                    wait_loads(sl_b2)
                    fire_gather(sl_b2)
                else:
                    @pl.when(g < NG - 1)
                    def _():
                        wait_loads(sl_b2)
                        fire_gather(sl_b2)

                if b < 2:
                    @pl.when(g > 0)
                    def _():
                        wait_scatters(sl_b4)
                    fire_loads(sl_b4, i + 4)
                else:
                    @pl.when(g < NG - 1)
                    def _():
                        wait_scatters(sl_b4)
                        fire_loads(sl_b4, i + 4)
            return 0

        lax.fori_loop(0, NG, outer, 0)
        for b in range(NSL):
            wait_scatters(slots[b])

        @pl.when(sid < NCHUNK - NS * MC1)
        def _():
            eb = pl.multiple_of((NS * MC1 + sid) * CH, 8)
            sl = slots[0]
            pltpu.sync_copy(src.at[pl.ds(eb, CH)], sl[0])
            pltpu.sync_copy(dst.at[pl.ds(eb, CH)], sl[1])
            pltpu.async_copy(nf_half.at[sl[0]], sl[2], sl[4]).wait()
            pltpu.sync_copy(sl[2], acc_n.at[sl[1]], add=True)
            if with_cnt:
                pltpu.sync_copy(msg_c, acc_c.at[sl[1]], add=True)

    @pl.when(cid == 0)
    def _():
        run_half(nf_lo, False)

    @pl.when(cid == 1)
    def _():
        run_half(nf_hi, True)

    plsc.subcore_barrier()

    pltpu.sync_copy(acc_n.at[pl.ds(row0, ROWS_PER_S)],
                    sn_out.at[cid, pl.ds(row0, ROWS_PER_S)])

    @pl.when(cid == 1)
    def _():
        pltpu.sync_copy(acc_c.at[pl.ds(row0, ROWS_PER_S)],
                        cnt_out.at[pl.ds(row0, ROWS_PER_S)])


_agg_a = pl.kernel(
    _agg_a_body,
    out_type=(
        jax.ShapeDtypeStruct((NC, N_PAD, DH), jnp.float32),
        jax.ShapeDtypeStruct((N_PAD, 16), jnp.float32),
    ),
    mesh=plsc.VectorSubcoreMesh(core_axis_name="c", subcore_axis_name="s"),
    compiler_params=pltpu.CompilerParams(
        use_tc_tiling_on_sc=False, needs_layout_passes=False),
    scratch_types=(
        [pltpu.VMEM((CH,), jnp.int32)] * 12
        + [pltpu.VMEM((CH, DH), jnp.float32)] * 6
        + [pltpu.VMEM((CH, 16), jnp.float32),
           pltpu.VMEM((128, DH), jnp.float32),
           pltpu.VMEM((ROWS_PER_S, 16), jnp.float32),
           pltpu.VMEM_SHARED((N_PAD, DH), jnp.float32),
           pltpu.VMEM_SHARED((N_PAD, 16), jnp.float32)]
        + [pltpu.SemaphoreType.DMA] * 18
    ),
)


def _agg_b_body(efeats, dst, se_out,
                id0, id1, id2, e0, e1, e2, zb_16, acc_e,
                ld0, ld1, ld2, sc0, sc1, sc2):
    cid = lax.axis_index("c")
    sid = lax.axis_index("s")
    w = cid * NS + sid

    _zero_vmem_2d(zb_16, ROWS_PER_S, 16)
    row0 = sid * ROWS_PER_S
    pltpu.sync_copy(zb_16, acc_e.at[pl.ds(row0, ROWS_PER_S)])
    plsc.subcore_barrier()

    cb = w * MC3
    slots = ((id0, e0, ld0, sc0), (id1, e1, ld1, sc1), (id2, e2, ld2, sc2))

    def fire_loads(sl, i):
        eb = pl.multiple_of((cb + i) * CH, 8)
        pltpu.async_copy(dst.at[pl.ds(eb, CH)], sl[0], sl[2])
        pltpu.async_copy(efeats.at[pl.ds(eb, CH)], sl[1], sl[2])

    def wait_loads(sl):
        pltpu.make_async_copy(dst.at[pl.ds(0, CH)], sl[0], sl[2]).wait()
        pltpu.make_async_copy(efeats.at[pl.ds(0, CH)], sl[1], sl[2]).wait()

    def fire_scatter(sl):
        pltpu.async_copy(sl[1], acc_e.at[sl[0]], sl[3], add=True)

    def wait_scatter(sl):
        pltpu.make_async_copy(sl[1], acc_e.at[sl[0]], sl[3]).wait()

    fire_loads(slots[0], 0)
    fire_loads(slots[1], 1)

    NG = MC3 // 3

    def outer(g, _):
        for b in range(3):
            i = g * 3 + b
            sl = slots[b]
            sl_nn = slots[(b + 2) % 3]

            wait_loads(sl)
            fire_scatter(sl)

            if b == 0:
                @pl.when(g > 0)
                def _():
                    wait_scatter(sl_nn)
                fire_loads(sl_nn, i + 2)
            else:
                @pl.when(g < NG - 1)
                def _():
                    wait_scatter(sl_nn)
                    fire_loads(sl_nn, i + 2)
        return 0

    lax.fori_loop(0, NG, outer, 0)
    for b in range(3):
        wait_scatter(slots[b])

    @pl.when(w < NCHUNK - NW * MC3)
    def _():
        eb = pl.multiple_of((NW * MC3 + w) * CH, 8)
        sl = slots[0]
        pltpu.sync_copy(dst.at[pl.ds(eb, CH)], sl[0])
        pltpu.sync_copy(efeats.at[pl.ds(eb, CH)], sl[1])
        pltpu.sync_copy(sl[1], acc_e.at[sl[0]], add=True)

    plsc.subcore_barrier()
    pltpu.sync_copy(acc_e.at[pl.ds(row0, ROWS_PER_S)],
                    se_out.at[cid, pl.ds(row0, ROWS_PER_S)])


_agg_b = pl.kernel(
    _agg_b_body,
    out_type=jax.ShapeDtypeStruct((NC, N_PAD, 16), jnp.float32),
    mesh=plsc.VectorSubcoreMesh(core_axis_name="c", subcore_axis_name="s"),
    compiler_params=pltpu.CompilerParams(
        use_tc_tiling_on_sc=False, needs_layout_passes=False),
    scratch_types=(
        [pltpu.VMEM((CH,), jnp.int32)] * 3
        + [pltpu.VMEM((CH, DE), jnp.float32)] * 3
        + [pltpu.VMEM((ROWS_PER_S, 16), jnp.float32),
           pltpu.VMEM_SHARED((N_PAD, 16), jnp.float32)]
        + [pltpu.SemaphoreType.DMA] * 6
    ),
)


def _node_mlp_body(snp, sep, cntp, nf, w1t, w2t, w3t, b, wst, wdt, be,
                   h_ref, ps_ref, pd_ref):
    sn = jnp.concatenate([snp[0], snp[1]], axis=1)
    se = sep[0] + sep[1]
    cnt = cntp[:, 0:1]
    inv = 1.0 / jnp.maximum(cnt, 1.0)
    t = (jnp.dot(sn, w2t[...], preferred_element_type=jnp.float32)
         + jnp.dot(se, w3t[...], preferred_element_type=jnp.float32))
    h = jnp.maximum(
        jnp.dot(nf[...], w1t[...], preferred_element_type=jnp.float32)
        + t * inv + b[...], 0.0)
    h_ref[...] = h
    ps_ref[...] = jnp.dot(h, wst[...], preferred_element_type=jnp.float32) + be[...]
    pd_ref[...] = jnp.dot(h, wdt[...], preferred_element_type=jnp.float32)


_BLK = 400
_node_mlp = pl.pallas_call(
    _node_mlp_body,
    grid=(N // _BLK,),
    in_specs=[
        pl.BlockSpec((NC, _BLK, DH), lambda i: (0, i, 0)),
        pl.BlockSpec((NC, _BLK, 16), lambda i: (0, i, 0)),
        pl.BlockSpec((_BLK, 16), lambda i: (i, 0)),
        pl.BlockSpec((_BLK, DN), lambda i: (i, 0)),
        pl.BlockSpec((DN, DN), lambda i: (0, 0)),
        pl.BlockSpec((DN, DN), lambda i: (0, 0)),
        pl.BlockSpec((DE, DN), lambda i: (0, 0)),
        pl.BlockSpec((1, DN), lambda i: (0, 0)),
        pl.BlockSpec((DN, DE), lambda i: (0, 0)),
        pl.BlockSpec((DN, DE), lambda i: (0, 0)),
        pl.BlockSpec((1, DE), lambda i: (0, 0)),
    ],
    out_specs=[
        pl.BlockSpec((_BLK, DN), lambda i: (i, 0)),
        pl.BlockSpec((_BLK, DE), lambda i: (i, 0)),
        pl.BlockSpec((_BLK, DE), lambda i: (i, 0)),
    ],
    out_shape=[
        jax.ShapeDtypeStruct((N, DN), jnp.float32),
        jax.ShapeDtypeStruct((N, DE), jnp.float32),
        jax.ShapeDtypeStruct((N, DE), jnp.float32),
    ],
)


def _edge_body(psrc, pdst, src, dst, out,
               is0, is1, is2, is3, is4, is5,
               id0, id1, id2, id3, id4, id5,
               a0, a1, a2, a3, a4, a5,
               b0, b1, b2, b3, b4, b5,
               o0, o1, o2, o3, o4, o5,
               ld0, ld1, ld2, ld3, ld4, ld5,
               sg0, sg1, sg2, sg3, sg4, sg5,
               st0, st1, st2, st3, st4, st5):
    cid = lax.axis_index("c")
    sid = lax.axis_index("s")
    w = cid * NS + sid
    cb = w * MC3

    slots = ((is0, id0, a0, b0, o0, ld0, sg0, st0),
             (is1, id1, a1, b1, o1, ld1, sg1, st1),
             (is2, id2, a2, b2, o2, ld2, sg2, st2),
             (is3, id3, a3, b3, o3, ld3, sg3, st3),
             (is4, id4, a4, b4, o4, ld4, sg4, st4),
             (is5, id5, a5, b5, o5, ld5, sg5, st5))
    NSL = 6

    def fire_loads(sl, i):
        eb = pl.multiple_of((cb + i) * CH, 8)
        pltpu.async_copy(src.at[pl.ds(eb, CH)], sl[0], sl[5])
        pltpu.async_copy(dst.at[pl.ds(eb, CH)], sl[1], sl[5])

    def wait_loads(sl):
        pltpu.make_async_copy(src.at[pl.ds(0, CH)], sl[0], sl[5]).wait()
        pltpu.make_async_copy(dst.at[pl.ds(0, CH)], sl[1], sl[5]).wait()

    def fire_gathers(sl):
        pltpu.async_copy(psrc.at[sl[0]], sl[2], sl[6])
        pltpu.async_copy(pdst.at[sl[1]], sl[3], sl[6])

    def wait_gathers(sl):
        pltpu.make_async_copy(psrc.at[sl[0]], sl[2], sl[6]).wait()
        pltpu.make_async_copy(pdst.at[sl[1]], sl[3], sl[6]).wait()

    def compute(sl):
        zeros16 = jnp.zeros((16,), jnp.int32)

        def body(j, iv):
            for r in range(4):
                jj = j * 4 + r
                v = jnp.maximum(sl[2][jj, :] + sl[3][jj, :], 0.0)
                plsc.store_scatter(sl[4], [zeros16, iv], v)
                iv = iv + 1
            return iv
        lax.fori_loop(0, CH // 4, body, lax.iota(jnp.int32, 16) * CH)

    def fire_store(sl, i):
        eb = pl.multiple_of((cb + i) * CH, 8)
        pltpu.async_copy(sl[4], out.at[:, pl.ds(eb, CH)], sl[7])

    def wait_store(sl):
        pltpu.make_async_copy(sl[4], out.at[:, pl.ds(0, CH)], sl[7]).wait()

    for i in range(4):
        fire_loads(slots[i], i)
    for i in range(2):
        wait_loads(slots[i])
        fire_gathers(slots[i])

    NG = MC3 // NSL

    def outer(g, _):
        for b in range(NSL):
            i = g * NSL + b
            sl = slots[b]
            sl_b2 = slots[(b + 2) % NSL]
            sl_b4 = slots[(b + 4) % NSL]

            wait_gathers(sl)

            @pl.when(g > 0)
            def _():
                wait_store(sl)
            compute(sl)
            fire_store(sl, i)

            if b < 4:
                wait_loads(sl_b2)
                fire_gathers(sl_b2)
            else:
                @pl.when(g < NG - 1)
                def _():
                    wait_loads(sl_b2)
                    fire_gathers(sl_b2)

            if b < 2:
                fire_loads(sl_b4, i + 4)
            else:
                @pl.when(g < NG - 1)
                def _():
                    fire_loads(sl_b4, i + 4)
        return 0

    lax.fori_loop(0, NG, outer, 0)
    for b in range(NSL):
        wait_store(slots[b])

    @pl.when(w < NCHUNK - NW * MC3)
    def _():
        eb = pl.multiple_of((NW * MC3 + w) * CH, 8)
        sl = slots[0]
        pltpu.sync_copy(src.at[pl.ds(eb, CH)], sl[0])
        pltpu.sync_copy(dst.at[pl.ds(eb, CH)], sl[1])
        ca = pltpu.async_copy(psrc.at[sl[0]], sl[2], sl[6])
        cbd = pltpu.async_copy(pdst.at[sl[1]], sl[3], sl[6])
        ca.wait()
        cbd.wait()
        compute(sl)
        pltpu.sync_copy(sl[4], out.at[:, pl.ds(eb, CH)])


_edge_mlp = pl.kernel(
    _edge_body,
    out_type=jax.ShapeDtypeStruct((DE, E), jnp.float32),
    mesh=plsc.VectorSubcoreMesh(core_axis_name="c", subcore_axis_name="s"),
    compiler_params=pltpu.CompilerParams(
        use_tc_tiling_on_sc=False, needs_layout_passes=False),
    scratch_types=(
        [pltpu.VMEM((CH,), jnp.int32)] * 12
        + [pltpu.VMEM((CH, DE), jnp.float32)] * 12
        + [pltpu.VMEM((DE, CH), jnp.float32)] * 6
        + [pltpu.SemaphoreType.DMA] * 18
    ),
)


def kernel(nfeats, efeats, edge_index, W_apply_w, W_apply_b, W_edge_w, W_edge_b):
    src = edge_index[0].astype(jnp.int32)
    dst = edge_index[1].astype(jnp.int32)

    sn_p, cnt_p = _agg_a(nfeats[:, :DH], nfeats[:, DH:], src, dst)
    se_p = _agg_b(efeats, dst)

    w1t = W_apply_w[:, :DN].T
    w2t = W_apply_w[:, DN:2 * DN].T
    w3t = W_apply_w[:, 2 * DN:].T
    wst = W_edge_w[:, :DN].T
    wdt = W_edge_w[:, DN:].T
    h_nodes, psrc, pdst = _node_mlp(
        sn_p, se_p, cnt_p, nfeats, w1t, w2t, w3t, W_apply_b[None, :],
        wst, wdt, W_edge_b[None, :])

    h_edges_t = _edge_mlp(psrc, pdst, src, dst)
    return (h_nodes, h_edges_t.T)
